# baseline (speedup 1.0000x reference)
import functools

import jax
import jax.numpy as jnp
from jax import lax
from jax.experimental import pallas as pl
from jax.experimental.pallas import tpu as pltpu
from jax.experimental.pallas import tpu_sc as plsc

_NUM_EXPERTS = 64
_TOP_K = 2
_INPUT_SIZE = 4096
_TOKENS = 8192
_BT = 256
_NB = _TOKENS // _BT
_NW = 32
_CH = _TOKENS * _TOP_K // _NW


def _tc_body(x_ref, w_ref, logits_ref, idx_ref, gates_ref, rank_ref,
             hist_ref, acc_ref):
    b = pl.program_id(0)

    @pl.when(b == 0)
    def _():
        acc_ref[...] = jnp.zeros_like(acc_ref)

    x = x_ref[...]
    w = w_ref[...]
    logits = lax.dot_general(x, w, (((1,), (1,)), ((), ())),
                             preferred_element_type=jnp.float32)
    logits_ref[...] = logits

    col = lax.broadcasted_iota(jnp.int32, (_BT, _NUM_EXPERTS), 1)
    m0 = jnp.max(logits, axis=1, keepdims=True)
    idx0 = jnp.min(jnp.where(logits == m0, col, _NUM_EXPERTS), axis=1,
                   keepdims=True)
    masked = jnp.where(col == idx0, -jnp.inf, logits)
    m1 = jnp.max(masked, axis=1, keepdims=True)
    idx1 = jnp.min(jnp.where(masked == m1, col, _NUM_EXPERTS), axis=1,
                   keepdims=True)

    s = jnp.exp(m1 - m0)
    inv = 1.0 / (1.0 + s)
    idx_ref[...] = jnp.concatenate([idx0, idx1], axis=1)
    gates_ref[...] = jnp.concatenate([inv, s * inv], axis=1)

    oh0 = (col == idx0).astype(jnp.float32)
    oh1 = (col == idx1).astype(jnp.float32)
    ohs = oh0 + oh1
    ri = lax.broadcasted_iota(jnp.int32, (_BT, _BT), 0)
    ci = lax.broadcasted_iota(jnp.int32, (_BT, _BT), 1)
    tril = (ci < ri).astype(jnp.float32)
    P = lax.dot_general(tril, ohs, (((1,), (0,)), ((), ())),
                        preferred_element_type=jnp.float32)
    acc = acc_ref[0:1, :]
    pa = P + acc
    base0 = jnp.sum(pa * oh0, axis=1, keepdims=True)
    base1 = jnp.sum(pa * oh1, axis=1, keepdims=True)
    rank_ref[...] = jnp.concatenate([base0, base1], axis=1).astype(jnp.int32)

    acc_new = acc + jnp.sum(ohs, axis=0, keepdims=True)
    acc_ref[...] = jnp.broadcast_to(acc_new, acc_ref.shape)
    hist_ref[...] = jnp.broadcast_to(acc_new, hist_ref.shape).astype(jnp.int32)


def _tc_call(x, w):
    out_shapes = [
        jax.ShapeDtypeStruct((_TOKENS, _NUM_EXPERTS), jnp.float32),
        jax.ShapeDtypeStruct((_TOKENS, _TOP_K), jnp.int32),
        jax.ShapeDtypeStruct((_TOKENS, _TOP_K), jnp.float32),
        jax.ShapeDtypeStruct((_TOKENS, _TOP_K), jnp.int32),
        jax.ShapeDtypeStruct((8, _NUM_EXPERTS), jnp.int32),
    ]
    in_specs = [
        pl.BlockSpec((_BT, _INPUT_SIZE), lambda b: (b, 0)),
        pl.BlockSpec((_NUM_EXPERTS, _INPUT_SIZE), lambda b: (0, 0)),
    ]
    out_specs = [
        pl.BlockSpec((_BT, _NUM_EXPERTS), lambda b: (b, 0)),
        pl.BlockSpec((_BT, _TOP_K), lambda b: (b, 0)),
        pl.BlockSpec((_BT, _TOP_K), lambda b: (b, 0)),
        pl.BlockSpec((_BT, _TOP_K), lambda b: (b, 0)),
        pl.BlockSpec((8, _NUM_EXPERTS), lambda b: (0, 0)),
    ]
    return pl.pallas_call(
        _tc_body,
        grid=(_NB,),
        in_specs=in_specs,
        out_specs=out_specs,
        out_shape=out_shapes,
        scratch_shapes=[pltpu.VMEM((8, _NUM_EXPERTS), jnp.float32)],
        compiler_params=pltpu.CompilerParams(
            dimension_semantics=("arbitrary",)),
    )(x, w)


def _sc_body(keys_hbm, r_hbm, g_hbm, hist_hbm,
             oi_hbm, ob_hbm, og_hbm,
             hist_v, excl_v, keys_v, r_v, g_v, dest_v, vals_v, bvals_v, sem):
    nc = 2
    wid = lax.axis_index("s") * nc + lax.axis_index("c")
    base = wid * _CH

    loads = [
        pltpu.make_async_copy(keys_hbm.at[pl.ds(base, _CH)], keys_v, sem),
        pltpu.make_async_copy(r_hbm.at[pl.ds(base, _CH)], r_v, sem),
        pltpu.make_async_copy(g_hbm.at[pl.ds(base, _CH)], g_v, sem),
        pltpu.make_async_copy(hist_hbm, hist_v, sem),
    ]
    for c in loads:
        c.start()
    for c in loads:
        c.wait()

    carry = jnp.int32(0)
    for j in range(4):
        c = hist_v[pl.ds(j * 16, 16)]
        s = plsc.cumsum(c)
        excl_v[pl.ds(j * 16, 16)] = s - c + carry
        carry = carry + jnp.sum(c)

    lanes = lax.iota(jnp.int32, 16)
    for i in range(_CH // 16):
        off = i * 16
        e = keys_v[pl.ds(off, 16)]
        eb = plsc.load_gather(excl_v, [e])
        d = eb + r_v[pl.ds(off, 16)]
        dest_v[pl.ds(off, 16)] = d
        gi = base + off + lanes
        vals_v[pl.ds(off, 16)] = gi
        bvals_v[pl.ds(off, 16)] = gi // 2

    stores = [
        pltpu.make_async_copy(vals_v, oi_hbm.at[dest_v], sem),
        pltpu.make_async_copy(bvals_v, ob_hbm.at[dest_v], sem),
        pltpu.make_async_copy(g_v, og_hbm.at[dest_v], sem),
    ]
    for c in stores:
        c.start()
    for c in stores:
        c.wait()


def _sc_call(keys, ranks, gates, hist):
    n = _TOKENS * _TOP_K
    mesh = plsc.VectorSubcoreMesh(core_axis_name="c", subcore_axis_name="s")
    f = pl.kernel(
        _sc_body,
        out_type=[
            jax.ShapeDtypeStruct((n,), jnp.int32),
            jax.ShapeDtypeStruct((n,), jnp.int32),
            jax.ShapeDtypeStruct((n,), jnp.float32),
        ],
        mesh=mesh,
        scratch_types=[
            pltpu.VMEM((_NUM_EXPERTS,), jnp.int32),
            pltpu.VMEM((_NUM_EXPERTS,), jnp.int32),
            pltpu.VMEM((_CH,), jnp.int32),
            pltpu.VMEM((_CH,), jnp.int32),
            pltpu.VMEM((_CH,), jnp.float32),
            pltpu.VMEM((_CH,), jnp.int32),
            pltpu.VMEM((_CH,), jnp.int32),
            pltpu.VMEM((_CH,), jnp.int32),
            pltpu.SemaphoreType.DMA,
        ],
        compiler_params=pltpu.CompilerParams(needs_layout_passes=False),
    )
    return f(keys, ranks, gates, hist)


def kernel(hidden_states, W):
    logits, idx, gates, rank, hist8 = _tc_call(hidden_states, W)
    expert_size = hist8[0]
    keys = idx.reshape(-1)
    r = rank.reshape(-1)
    g = gates.reshape(-1)
    isorted, bidx, bgates = _sc_call(keys, r, g, expert_size)
    return (isorted, bidx, bgates, expert_size, logits)


# speedup vs baseline: 1.5112x; 1.5112x over previous
import functools

import jax
import jax.numpy as jnp
from jax import lax
from jax.experimental import pallas as pl
from jax.experimental.pallas import tpu as pltpu
from jax.experimental.pallas import tpu_sc as plsc

_NUM_EXPERTS = 64
_TOP_K = 2
_INPUT_SIZE = 4096
_TOKENS = 8192
_BT = 256
_NB = _TOKENS // _BT
_NW = 32
_CH = _TOKENS * _TOP_K // _NW


def _tc_body(x_ref, w_ref, logits_ref, idx_ref, gates_ref, rank_ref,
             hist_ref, acc_ref):
    b = pl.program_id(0)

    @pl.when(b == 0)
    def _():
        acc_ref[...] = jnp.zeros_like(acc_ref)

    x = x_ref[...]
    w = w_ref[...]
    logits = lax.dot_general(x, w, (((1,), (1,)), ((), ())),
                             preferred_element_type=jnp.float32)
    logits_ref[...] = logits

    col = lax.broadcasted_iota(jnp.int32, (_BT, _NUM_EXPERTS), 1)
    m0 = jnp.max(logits, axis=1, keepdims=True)
    idx0 = jnp.min(jnp.where(logits == m0, col, _NUM_EXPERTS), axis=1,
                   keepdims=True)
    masked = jnp.where(col == idx0, -jnp.inf, logits)
    m1 = jnp.max(masked, axis=1, keepdims=True)
    idx1 = jnp.min(jnp.where(masked == m1, col, _NUM_EXPERTS), axis=1,
                   keepdims=True)

    s = jnp.exp(m1 - m0)
    inv = 1.0 / (1.0 + s)
    idx_ref[...] = jnp.concatenate([idx0, idx1], axis=1)
    gates_ref[...] = jnp.concatenate([inv, s * inv], axis=1)

    oh0 = (col == idx0).astype(jnp.float32)
    oh1 = (col == idx1).astype(jnp.float32)
    ohs = oh0 + oh1
    ri = lax.broadcasted_iota(jnp.int32, (_BT, _BT), 0)
    ci = lax.broadcasted_iota(jnp.int32, (_BT, _BT), 1)
    tril = (ci < ri).astype(jnp.float32)
    P = lax.dot_general(tril, ohs, (((1,), (0,)), ((), ())),
                        preferred_element_type=jnp.float32)
    acc = acc_ref[0:1, :]
    pa = P + acc
    base0 = jnp.sum(pa * oh0, axis=1, keepdims=True)
    base1 = jnp.sum(pa * oh1, axis=1, keepdims=True)
    rank_ref[...] = jnp.concatenate([base0, base1], axis=1).astype(jnp.int32)

    acc_new = acc + jnp.sum(ohs, axis=0, keepdims=True)
    acc_ref[...] = jnp.broadcast_to(acc_new, acc_ref.shape)
    hist_ref[...] = jnp.broadcast_to(acc_new, hist_ref.shape).astype(jnp.int32)


def _tc_call(x, w):
    out_shapes = [
        jax.ShapeDtypeStruct((_TOKENS, _NUM_EXPERTS), jnp.float32),
        jax.ShapeDtypeStruct((_TOKENS, _TOP_K), jnp.int32),
        jax.ShapeDtypeStruct((_TOKENS, _TOP_K), jnp.float32),
        jax.ShapeDtypeStruct((_TOKENS, _TOP_K), jnp.int32),
        jax.ShapeDtypeStruct((8, _NUM_EXPERTS), jnp.int32),
    ]
    in_specs = [
        pl.BlockSpec((_BT, _INPUT_SIZE), lambda b: (b, 0)),
        pl.BlockSpec((_NUM_EXPERTS, _INPUT_SIZE), lambda b: (0, 0)),
    ]
    out_specs = [
        pl.BlockSpec((_BT, _NUM_EXPERTS), lambda b: (b, 0)),
        pl.BlockSpec((_BT, _TOP_K), lambda b: (b, 0)),
        pl.BlockSpec((_BT, _TOP_K), lambda b: (b, 0)),
        pl.BlockSpec((_BT, _TOP_K), lambda b: (b, 0)),
        pl.BlockSpec((8, _NUM_EXPERTS), lambda b: (0, 0)),
    ]
    return pl.pallas_call(
        _tc_body,
        grid=(_NB,),
        in_specs=in_specs,
        out_specs=out_specs,
        out_shape=out_shapes,
        scratch_shapes=[pltpu.VMEM((8, _NUM_EXPERTS), jnp.float32)],
        compiler_params=pltpu.CompilerParams(
            dimension_semantics=("arbitrary",)),
    )(x, w)


def _sc_body(keys_hbm, r_hbm, g_hbm, hist_hbm,
             oi_hbm, ob_hbm, og_hbm,
             hist_v, excl_v, keys_v, r_v, g_v, inv_v, ob_v, og_v, sem):
    n = _TOKENS * _TOP_K
    nc = 2
    wid = lax.axis_index("s") * nc + lax.axis_index("c")
    base = wid * _CH

    loads = [
        pltpu.make_async_copy(keys_hbm, keys_v, sem),
        pltpu.make_async_copy(r_hbm, r_v, sem),
        pltpu.make_async_copy(g_hbm, g_v, sem),
        pltpu.make_async_copy(hist_hbm, hist_v, sem),
    ]
    for c in loads:
        c.start()
    for c in loads:
        c.wait()

    carry = jnp.int32(0)
    for j in range(4):
        c = hist_v[pl.ds(j * 16, 16)]
        s = plsc.cumsum(c)
        excl_v[pl.ds(j * 16, 16)] = s - c + carry
        carry = carry + jnp.sum(c)

    lanes = lax.iota(jnp.int32, 16)

    def scan_body(i, _):
        off = i * 16
        k = keys_v[pl.ds(off, 16)]
        r = r_v[pl.ds(off, 16)]
        d = plsc.load_gather(excl_v, [k]) + r
        rel = d - base
        m = (rel >= 0) & (rel < _CH)
        plsc.store_scatter(inv_v, [rel & (_CH - 1)], off + lanes, mask=m)
        return 0

    lax.fori_loop(0, n // 16, scan_body, 0)

    for i in range(_CH // 16):
        off = i * 16
        inv = inv_v[pl.ds(off, 16)]
        ob_v[pl.ds(off, 16)] = inv >> 1
        og_v[pl.ds(off, 16)] = plsc.load_gather(g_v, [inv])

    stores = [
        pltpu.make_async_copy(inv_v, oi_hbm.at[pl.ds(base, _CH)], sem),
        pltpu.make_async_copy(ob_v, ob_hbm.at[pl.ds(base, _CH)], sem),
        pltpu.make_async_copy(og_v, og_hbm.at[pl.ds(base, _CH)], sem),
    ]
    for c in stores:
        c.start()
    for c in stores:
        c.wait()


def _sc_call(keys, ranks, gates, hist):
    n = _TOKENS * _TOP_K
    mesh = plsc.VectorSubcoreMesh(core_axis_name="c", subcore_axis_name="s")
    f = pl.kernel(
        _sc_body,
        out_type=[
            jax.ShapeDtypeStruct((n,), jnp.int32),
            jax.ShapeDtypeStruct((n,), jnp.int32),
            jax.ShapeDtypeStruct((n,), jnp.float32),
        ],
        mesh=mesh,
        scratch_types=[
            pltpu.VMEM((_NUM_EXPERTS,), jnp.int32),
            pltpu.VMEM((_NUM_EXPERTS,), jnp.int32),
            pltpu.VMEM((n,), jnp.int32),
            pltpu.VMEM((n,), jnp.int32),
            pltpu.VMEM((n,), jnp.float32),
            pltpu.VMEM((_CH,), jnp.int32),
            pltpu.VMEM((_CH,), jnp.int32),
            pltpu.VMEM((_CH,), jnp.float32),
            pltpu.SemaphoreType.DMA,
        ],
        compiler_params=pltpu.CompilerParams(needs_layout_passes=False),
    )
    return f(keys, ranks, gates, hist)


def kernel(hidden_states, W):
    logits, idx, gates, rank, hist8 = _tc_call(hidden_states, W)
    expert_size = hist8[0]
    keys = idx.reshape(-1)
    r = rank.reshape(-1)
    g = gates.reshape(-1)
    isorted, bidx, bgates = _sc_call(keys, r, g, expert_size)
    return (isorted, bidx, bgates, expert_size, logits)


# speedup vs baseline: 1.7092x; 1.1310x over previous
import functools

import jax
import jax.numpy as jnp
from jax import lax
from jax.experimental import pallas as pl
from jax.experimental.pallas import tpu as pltpu
from jax.experimental.pallas import tpu_sc as plsc

_NUM_EXPERTS = 64
_TOP_K = 2
_INPUT_SIZE = 4096
_TOKENS = 8192
_BT = 512
_NB = _TOKENS // _BT
_NW = 32
_CH = _TOKENS * _TOP_K // _NW


def _tc_body(x_ref, w_ref, logits_ref, idx_ref, gates_ref, rank_ref,
             hist_ref, acc_ref):
    b = pl.program_id(0)

    @pl.when(b == 0)
    def _():
        acc_ref[...] = jnp.zeros_like(acc_ref)

    x = x_ref[...]
    w = w_ref[...]
    logits = lax.dot_general(x, w, (((1,), (1,)), ((), ())),
                             preferred_element_type=jnp.float32)
    logits_ref[...] = logits

    col = lax.broadcasted_iota(jnp.int32, (_BT, _NUM_EXPERTS), 1)
    m0 = jnp.max(logits, axis=1, keepdims=True)
    idx0 = jnp.min(jnp.where(logits == m0, col, _NUM_EXPERTS), axis=1,
                   keepdims=True)
    masked = jnp.where(col == idx0, -jnp.inf, logits)
    m1 = jnp.max(masked, axis=1, keepdims=True)
    idx1 = jnp.min(jnp.where(masked == m1, col, _NUM_EXPERTS), axis=1,
                   keepdims=True)

    s = jnp.exp(m1 - m0)
    inv = 1.0 / (1.0 + s)
    idx_ref[...] = jnp.concatenate([idx0, idx1], axis=1)
    gates_ref[...] = jnp.concatenate([inv, s * inv], axis=1)

    oh0 = (col == idx0).astype(jnp.float32)
    oh1 = (col == idx1).astype(jnp.float32)
    ohs = oh0 + oh1
    ri = lax.broadcasted_iota(jnp.int32, (_BT, _BT), 0)
    ci = lax.broadcasted_iota(jnp.int32, (_BT, _BT), 1)
    tril = (ci < ri).astype(jnp.float32)
    P = lax.dot_general(tril, ohs, (((1,), (0,)), ((), ())),
                        preferred_element_type=jnp.float32)
    acc = acc_ref[0:1, :]
    pa = P + acc
    base0 = jnp.sum(pa * oh0, axis=1, keepdims=True)
    base1 = jnp.sum(pa * oh1, axis=1, keepdims=True)
    rank_ref[...] = jnp.concatenate([base0, base1], axis=1).astype(jnp.int32)

    acc_new = acc + jnp.sum(ohs, axis=0, keepdims=True)
    acc_ref[...] = jnp.broadcast_to(acc_new, acc_ref.shape)
    hist_ref[...] = jnp.broadcast_to(acc_new, hist_ref.shape).astype(jnp.int32)


def _tc_call(x, w):
    out_shapes = [
        jax.ShapeDtypeStruct((_TOKENS, _NUM_EXPERTS), jnp.float32),
        jax.ShapeDtypeStruct((_TOKENS, _TOP_K), jnp.int32),
        jax.ShapeDtypeStruct((_TOKENS, _TOP_K), jnp.float32),
        jax.ShapeDtypeStruct((_TOKENS, _TOP_K), jnp.int32),
        jax.ShapeDtypeStruct((8, _NUM_EXPERTS), jnp.int32),
    ]
    in_specs = [
        pl.BlockSpec((_BT, _INPUT_SIZE), lambda b: (b, 0)),
        pl.BlockSpec((_NUM_EXPERTS, _INPUT_SIZE), lambda b: (0, 0)),
    ]
    out_specs = [
        pl.BlockSpec((_BT, _NUM_EXPERTS), lambda b: (b, 0)),
        pl.BlockSpec((_BT, _TOP_K), lambda b: (b, 0)),
        pl.BlockSpec((_BT, _TOP_K), lambda b: (b, 0)),
        pl.BlockSpec((_BT, _TOP_K), lambda b: (b, 0)),
        pl.BlockSpec((8, _NUM_EXPERTS), lambda b: (0, 0)),
    ]
    return pl.pallas_call(
        _tc_body,
        grid=(_NB,),
        in_specs=in_specs,
        out_specs=out_specs,
        out_shape=out_shapes,
        scratch_shapes=[pltpu.VMEM((8, _NUM_EXPERTS), jnp.float32)],
        compiler_params=pltpu.CompilerParams(
            dimension_semantics=("arbitrary",)),
    )(x, w)


def _sc_body(keys_hbm, r_hbm, g_hbm, hist_hbm,
             oi_hbm, ob_hbm, og_hbm,
             hist_v, excl_v, keys_v, r_v, g_v, inv_v, ob_v, og_v, sem):
    n = _TOKENS * _TOP_K
    nc = 2
    wid = lax.axis_index("s") * nc + lax.axis_index("c")
    base = wid * _CH

    loads = [
        pltpu.make_async_copy(keys_hbm, keys_v, sem),
        pltpu.make_async_copy(r_hbm, r_v, sem),
        pltpu.make_async_copy(hist_hbm, hist_v, sem),
    ]
    gload = pltpu.make_async_copy(g_hbm, g_v, sem)
    for c in loads:
        c.start()
    gload.start()
    for c in loads:
        c.wait()

    carry = jnp.int32(0)
    for j in range(4):
        c = hist_v[pl.ds(j * 16, 16)]
        s = plsc.cumsum(c)
        excl_v[pl.ds(j * 16, 16)] = s - c + carry
        carry = carry + jnp.sum(c)

    lanes = lax.iota(jnp.int32, 16)

    def scan_body(i, _):
        for u in range(8):
            off = i * 128 + u * 16
            k = keys_v[pl.ds(off, 16)]
            r = r_v[pl.ds(off, 16)]
            d = plsc.load_gather(excl_v, [k]) + r
            rel = d - base
            m = (rel >= 0) & (rel < _CH)
            plsc.store_scatter(inv_v, [rel & (_CH - 1)], off + lanes, mask=m)
        return 0

    lax.fori_loop(0, n // 128, scan_body, 0)
    gload.wait()

    for i in range(_CH // 16):
        off = i * 16
        inv = inv_v[pl.ds(off, 16)]
        ob_v[pl.ds(off, 16)] = inv >> 1
        og_v[pl.ds(off, 16)] = plsc.load_gather(g_v, [inv])

    stores = [
        pltpu.make_async_copy(inv_v, oi_hbm.at[pl.ds(base, _CH)], sem),
        pltpu.make_async_copy(ob_v, ob_hbm.at[pl.ds(base, _CH)], sem),
        pltpu.make_async_copy(og_v, og_hbm.at[pl.ds(base, _CH)], sem),
    ]
    for c in stores:
        c.start()
    for c in stores:
        c.wait()


def _sc_call(keys, ranks, gates, hist):
    n = _TOKENS * _TOP_K
    mesh = plsc.VectorSubcoreMesh(core_axis_name="c", subcore_axis_name="s")
    f = pl.kernel(
        _sc_body,
        out_type=[
            jax.ShapeDtypeStruct((n,), jnp.int32),
            jax.ShapeDtypeStruct((n,), jnp.int32),
            jax.ShapeDtypeStruct((n,), jnp.float32),
        ],
        mesh=mesh,
        scratch_types=[
            pltpu.VMEM((_NUM_EXPERTS,), jnp.int32),
            pltpu.VMEM((_NUM_EXPERTS,), jnp.int32),
            pltpu.VMEM((n,), jnp.int32),
            pltpu.VMEM((n,), jnp.int32),
            pltpu.VMEM((n,), jnp.float32),
            pltpu.VMEM((_CH,), jnp.int32),
            pltpu.VMEM((_CH,), jnp.int32),
            pltpu.VMEM((_CH,), jnp.float32),
            pltpu.SemaphoreType.DMA,
        ],
        compiler_params=pltpu.CompilerParams(needs_layout_passes=False),
    )
    return f(keys, ranks, gates, hist)


def kernel(hidden_states, W):
    logits, idx, gates, rank, hist8 = _tc_call(hidden_states, W)
    expert_size = hist8[0]
    keys = idx.reshape(-1)
    r = rank.reshape(-1)
    g = gates.reshape(-1)
    isorted, bidx, bgates = _sc_call(keys, r, g, expert_size)
    return (isorted, bidx, bgates, expert_size, logits)


# speedup vs baseline: 1.8502x; 1.0825x over previous
import functools

import jax
import jax.numpy as jnp
from jax import lax
from jax.experimental import pallas as pl
from jax.experimental.pallas import tpu as pltpu
from jax.experimental.pallas import tpu_sc as plsc

_NUM_EXPERTS = 64
_TOP_K = 2
_INPUT_SIZE = 4096
_TOKENS = 8192
_BT = 512
_NB = _TOKENS // _BT
_NW = 32
_CH = _TOKENS * _TOP_K // _NW


def _tc_body(x_ref, w_ref, logits_ref, packed_ref, gates_ref,
             hist_ref, acc_ref):
    b = pl.program_id(0)

    @pl.when(b == 0)
    def _():
        acc_ref[...] = jnp.zeros_like(acc_ref)

    x = x_ref[...]
    w = w_ref[...]
    logits = lax.dot_general(x, w, (((1,), (1,)), ((), ())),
                             preferred_element_type=jnp.float32)
    logits_ref[...] = logits

    col = lax.broadcasted_iota(jnp.int32, (_BT, _NUM_EXPERTS), 1)
    m0 = jnp.max(logits, axis=1, keepdims=True)
    idx0 = jnp.min(jnp.where(logits == m0, col, _NUM_EXPERTS), axis=1,
                   keepdims=True)
    masked = jnp.where(col == idx0, -jnp.inf, logits)
    m1 = jnp.max(masked, axis=1, keepdims=True)
    idx1 = jnp.min(jnp.where(masked == m1, col, _NUM_EXPERTS), axis=1,
                   keepdims=True)

    s = jnp.exp(m1 - m0)
    inv = 1.0 / (1.0 + s)
    gates_ref[...] = jnp.concatenate([inv, s * inv], axis=1)

    oh0 = (col == idx0).astype(jnp.float32)
    oh1 = (col == idx1).astype(jnp.float32)
    ohs = oh0 + oh1
    ri = lax.broadcasted_iota(jnp.int32, (_BT, _BT), 0)
    ci = lax.broadcasted_iota(jnp.int32, (_BT, _BT), 1)
    tril = (ci < ri).astype(jnp.float32)
    P = lax.dot_general(tril, ohs, (((1,), (0,)), ((), ())),
                        preferred_element_type=jnp.float32)
    acc = acc_ref[0:1, :]
    pa = P + acc
    base0 = jnp.sum(pa * oh0, axis=1, keepdims=True).astype(jnp.int32)
    base1 = jnp.sum(pa * oh1, axis=1, keepdims=True).astype(jnp.int32)
    packed_ref[...] = jnp.concatenate(
        [idx0 * 16384 + base0, idx1 * 16384 + base1], axis=1)

    acc_new = acc + jnp.sum(ohs, axis=0, keepdims=True)
    acc_ref[...] = jnp.broadcast_to(acc_new, acc_ref.shape)
    hist_ref[...] = jnp.broadcast_to(acc_new, hist_ref.shape).astype(jnp.int32)


def _tc_call(x, w):
    out_shapes = [
        jax.ShapeDtypeStruct((_TOKENS, _NUM_EXPERTS), jnp.float32),
        jax.ShapeDtypeStruct((_TOKENS, _TOP_K), jnp.int32),
        jax.ShapeDtypeStruct((_TOKENS, _TOP_K), jnp.float32),
        jax.ShapeDtypeStruct((8, _NUM_EXPERTS), jnp.int32),
    ]
    in_specs = [
        pl.BlockSpec((_BT, _INPUT_SIZE), lambda b: (b, 0)),
        pl.BlockSpec((_NUM_EXPERTS, _INPUT_SIZE), lambda b: (0, 0)),
    ]
    out_specs = [
        pl.BlockSpec((_BT, _NUM_EXPERTS), lambda b: (b, 0)),
        pl.BlockSpec((_BT, _TOP_K), lambda b: (b, 0)),
        pl.BlockSpec((_BT, _TOP_K), lambda b: (b, 0)),
        pl.BlockSpec((8, _NUM_EXPERTS), lambda b: (0, 0)),
    ]
    return pl.pallas_call(
        _tc_body,
        grid=(_NB,),
        in_specs=in_specs,
        out_specs=out_specs,
        out_shape=out_shapes,
        scratch_shapes=[pltpu.VMEM((8, _NUM_EXPERTS), jnp.float32)],
        compiler_params=pltpu.CompilerParams(
            dimension_semantics=("arbitrary",)),
    )(x, w)


def _sc_body(p_hbm, g_hbm, hist_hbm,
             oi_hbm, ob_hbm, og_hbm,
             hist_v, excl_v, p_v, g_v, inv_v, ob_v, og_v, sem):
    n = _TOKENS * _TOP_K
    half = n // 2
    nc = 2
    wid = lax.axis_index("s") * nc + lax.axis_index("c")
    base = wid * _CH

    hload = pltpu.make_async_copy(hist_hbm, hist_v, sem)
    pload0 = pltpu.make_async_copy(
        p_hbm.at[pl.ds(0, half)], p_v.at[pl.ds(0, half)], sem)
    pload1 = pltpu.make_async_copy(
        p_hbm.at[pl.ds(half, half)], p_v.at[pl.ds(half, half)], sem)
    gload = pltpu.make_async_copy(g_hbm, g_v, sem)
    hload.start()
    pload0.start()
    pload1.start()
    gload.start()
    hload.wait()

    carry = jnp.int32(0)
    for j in range(4):
        c = hist_v[pl.ds(j * 16, 16)]
        s = plsc.cumsum(c)
        excl_v[pl.ds(j * 16, 16)] = s - c + carry
        carry = carry + jnp.sum(c)

    lanes = lax.iota(jnp.int32, 16)

    def scan_body(i, _):
        for u in range(8):
            off = i * 128 + u * 16
            p = p_v[pl.ds(off, 16)]
            k = lax.shift_right_logical(p, 14)
            r = p & 16383
            d = plsc.load_gather(excl_v, [k]) + r
            rel = d - base
            m = (rel >= 0) & (rel < _CH)
            plsc.store_scatter(inv_v, [rel & (_CH - 1)], off + lanes, mask=m)
        return 0

    pload0.wait()
    lax.fori_loop(0, n // 256, scan_body, 0)
    pload1.wait()
    lax.fori_loop(n // 256, n // 128, scan_body, 0)
    gload.wait()

    for i in range(_CH // 16):
        off = i * 16
        inv = inv_v[pl.ds(off, 16)]
        ob_v[pl.ds(off, 16)] = inv >> 1
        og_v[pl.ds(off, 16)] = plsc.load_gather(g_v, [inv])

    stores = [
        pltpu.make_async_copy(inv_v, oi_hbm.at[pl.ds(base, _CH)], sem),
        pltpu.make_async_copy(ob_v, ob_hbm.at[pl.ds(base, _CH)], sem),
        pltpu.make_async_copy(og_v, og_hbm.at[pl.ds(base, _CH)], sem),
    ]
    for c in stores:
        c.start()
    for c in stores:
        c.wait()


def _sc_call(packed, gates, hist):
    n = _TOKENS * _TOP_K
    mesh = plsc.VectorSubcoreMesh(core_axis_name="c", subcore_axis_name="s")
    f = pl.kernel(
        _sc_body,
        out_type=[
            jax.ShapeDtypeStruct((n,), jnp.int32),
            jax.ShapeDtypeStruct((n,), jnp.int32),
            jax.ShapeDtypeStruct((n,), jnp.float32),
        ],
        mesh=mesh,
        scratch_types=[
            pltpu.VMEM((_NUM_EXPERTS,), jnp.int32),
            pltpu.VMEM((_NUM_EXPERTS,), jnp.int32),
            pltpu.VMEM((n,), jnp.int32),
            pltpu.VMEM((n,), jnp.float32),
            pltpu.VMEM((_CH,), jnp.int32),
            pltpu.VMEM((_CH,), jnp.int32),
            pltpu.VMEM((_CH,), jnp.float32),
            pltpu.SemaphoreType.DMA,
        ],
        compiler_params=pltpu.CompilerParams(needs_layout_passes=False),
    )
    return f(packed, gates, hist)


def kernel(hidden_states, W):
    logits, packed, gates, hist8 = _tc_call(hidden_states, W)
    expert_size = hist8[0]
    isorted, bidx, bgates = _sc_call(
        packed.reshape(-1), gates.reshape(-1), expert_size)
    return (isorted, bidx, bgates, expert_size, logits)


# speedup vs baseline: 1.8754x; 1.0136x over previous
import functools

import jax
import jax.numpy as jnp
from jax import lax
from jax.experimental import pallas as pl
from jax.experimental.pallas import tpu as pltpu
from jax.experimental.pallas import tpu_sc as plsc

_NUM_EXPERTS = 64
_TOP_K = 2
_INPUT_SIZE = 4096
_TOKENS = 8192
_BT = 1024
_NB = _TOKENS // _BT
_NW = 32
_CH = _TOKENS * _TOP_K // _NW


def _tc_body(x_ref, w_ref, logits_ref, packed_ref, gates_ref,
             hist_ref, acc_ref):
    b = pl.program_id(0)

    @pl.when(b == 0)
    def _():
        acc_ref[...] = jnp.zeros_like(acc_ref)

    x = x_ref[...]
    w = w_ref[...]
    logits = lax.dot_general(x, w, (((1,), (1,)), ((), ())),
                             preferred_element_type=jnp.float32)
    logits_ref[...] = logits

    col = lax.broadcasted_iota(jnp.int32, (_BT, _NUM_EXPERTS), 1)
    m0 = jnp.max(logits, axis=1, keepdims=True)
    idx0 = jnp.min(jnp.where(logits == m0, col, _NUM_EXPERTS), axis=1,
                   keepdims=True)
    masked = jnp.where(col == idx0, -jnp.inf, logits)
    m1 = jnp.max(masked, axis=1, keepdims=True)
    idx1 = jnp.min(jnp.where(masked == m1, col, _NUM_EXPERTS), axis=1,
                   keepdims=True)

    s = jnp.exp(m1 - m0)
    inv = 1.0 / (1.0 + s)
    gates_ref[...] = jnp.concatenate([inv, s * inv], axis=1)

    oh0 = (col == idx0).astype(jnp.float32)
    oh1 = (col == idx1).astype(jnp.float32)
    ohs = oh0 + oh1
    ri = lax.broadcasted_iota(jnp.int32, (_BT, _BT), 0)
    ci = lax.broadcasted_iota(jnp.int32, (_BT, _BT), 1)
    tril = (ci < ri).astype(jnp.float32)
    P = lax.dot_general(tril, ohs, (((1,), (0,)), ((), ())),
                        preferred_element_type=jnp.float32)
    acc = acc_ref[0:1, :]
    pa = P + acc
    base0 = jnp.sum(pa * oh0, axis=1, keepdims=True).astype(jnp.int32)
    base1 = jnp.sum(pa * oh1, axis=1, keepdims=True).astype(jnp.int32)
    packed_ref[...] = jnp.concatenate(
        [idx0 * 16384 + base0, idx1 * 16384 + base1], axis=1)

    acc_new = acc + jnp.sum(ohs, axis=0, keepdims=True)
    acc_ref[...] = jnp.broadcast_to(acc_new, acc_ref.shape)
    hist_ref[...] = jnp.broadcast_to(acc_new, hist_ref.shape).astype(jnp.int32)


def _tc_call(x, w):
    out_shapes = [
        jax.ShapeDtypeStruct((_TOKENS, _NUM_EXPERTS), jnp.float32),
        jax.ShapeDtypeStruct((_TOKENS, _TOP_K), jnp.int32),
        jax.ShapeDtypeStruct((_TOKENS, _TOP_K), jnp.float32),
        jax.ShapeDtypeStruct((8, _NUM_EXPERTS), jnp.int32),
    ]
    in_specs = [
        pl.BlockSpec((_BT, _INPUT_SIZE), lambda b: (b, 0)),
        pl.BlockSpec((_NUM_EXPERTS, _INPUT_SIZE), lambda b: (0, 0)),
    ]
    out_specs = [
        pl.BlockSpec((_BT, _NUM_EXPERTS), lambda b: (b, 0)),
        pl.BlockSpec((_BT, _TOP_K), lambda b: (b, 0)),
        pl.BlockSpec((_BT, _TOP_K), lambda b: (b, 0)),
        pl.BlockSpec((8, _NUM_EXPERTS), lambda b: (0, 0)),
    ]
    return pl.pallas_call(
        _tc_body,
        grid=(_NB,),
        in_specs=in_specs,
        out_specs=out_specs,
        out_shape=out_shapes,
        scratch_shapes=[pltpu.VMEM((8, _NUM_EXPERTS), jnp.float32)],
        compiler_params=pltpu.CompilerParams(
            dimension_semantics=("arbitrary",)),
    )(x, w)


def _sc_body(p_hbm, g_hbm, hist_hbm,
             oi_hbm, ob_hbm, og_hbm,
             hist_v, excl_v, p_v, g_v, inv_v, ob_v, og_v, sem):
    n = _TOKENS * _TOP_K
    half = n // 2
    nc = 2
    wid = lax.axis_index("s") * nc + lax.axis_index("c")
    base = wid * _CH

    hload = pltpu.make_async_copy(hist_hbm, hist_v, sem)
    pload0 = pltpu.make_async_copy(
        p_hbm.at[pl.ds(0, half)], p_v.at[pl.ds(0, half)], sem)
    pload1 = pltpu.make_async_copy(
        p_hbm.at[pl.ds(half, half)], p_v.at[pl.ds(half, half)], sem)
    gload = pltpu.make_async_copy(g_hbm, g_v, sem)
    hload.start()
    pload0.start()
    pload1.start()
    gload.start()
    hload.wait()

    carry = jnp.int32(0)
    for j in range(4):
        c = hist_v[pl.ds(j * 16, 16)]
        s = plsc.cumsum(c)
        excl_v[pl.ds(j * 16, 16)] = s - c + carry
        carry = carry + jnp.sum(c)

    lanes = lax.iota(jnp.int32, 16)

    def scan_body(i, _):
        for u in range(8):
            off = i * 128 + u * 16
            p = p_v[pl.ds(off, 16)]
            k = lax.shift_right_logical(p, 14)
            r = p & 16383
            d = plsc.load_gather(excl_v, [k]) + r
            rel = d - base
            m = (rel >= 0) & (rel < _CH)
            plsc.store_scatter(inv_v, [rel & (_CH - 1)], off + lanes, mask=m)
        return 0

    pload0.wait()
    lax.fori_loop(0, n // 256, scan_body, 0)
    pload1.wait()
    lax.fori_loop(n // 256, n // 128, scan_body, 0)
    gload.wait()

    for i in range(_CH // 16):
        off = i * 16
        inv = inv_v[pl.ds(off, 16)]
        ob_v[pl.ds(off, 16)] = inv >> 1
        og_v[pl.ds(off, 16)] = plsc.load_gather(g_v, [inv])

    stores = [
        pltpu.make_async_copy(inv_v, oi_hbm.at[pl.ds(base, _CH)], sem),
        pltpu.make_async_copy(ob_v, ob_hbm.at[pl.ds(base, _CH)], sem),
        pltpu.make_async_copy(og_v, og_hbm.at[pl.ds(base, _CH)], sem),
    ]
    for c in stores:
        c.start()
    for c in stores:
        c.wait()


def _sc_call(packed, gates, hist):
    n = _TOKENS * _TOP_K
    mesh = plsc.VectorSubcoreMesh(core_axis_name="c", subcore_axis_name="s")
    f = pl.kernel(
        _sc_body,
        out_type=[
            jax.ShapeDtypeStruct((n,), jnp.int32),
            jax.ShapeDtypeStruct((n,), jnp.int32),
            jax.ShapeDtypeStruct((n,), jnp.float32),
        ],
        mesh=mesh,
        scratch_types=[
            pltpu.VMEM((_NUM_EXPERTS,), jnp.int32),
            pltpu.VMEM((_NUM_EXPERTS,), jnp.int32),
            pltpu.VMEM((n,), jnp.int32),
            pltpu.VMEM((n,), jnp.float32),
            pltpu.VMEM((_CH,), jnp.int32),
            pltpu.VMEM((_CH,), jnp.int32),
            pltpu.VMEM((_CH,), jnp.float32),
            pltpu.SemaphoreType.DMA,
        ],
        compiler_params=pltpu.CompilerParams(needs_layout_passes=False),
    )
    return f(packed, gates, hist)


def kernel(hidden_states, W):
    logits, packed, gates, hist8 = _tc_call(hidden_states, W)
    expert_size = hist8[0]
    isorted, bidx, bgates = _sc_call(
        packed.reshape(-1), gates.reshape(-1), expert_size)
    return (isorted, bidx, bgates, expert_size, logits)


# speedup vs baseline: 2.2434x; 1.1962x over previous
import functools

import jax
import jax.numpy as jnp
from jax import lax
from jax.experimental import pallas as pl
from jax.experimental.pallas import tpu as pltpu
from jax.experimental.pallas import tpu_sc as plsc

_NUM_EXPERTS = 64
_TOP_K = 2
_INPUT_SIZE = 4096
_TOKENS = 8192
_BT = 1024
_NB = _TOKENS // _BT
_NW = 32
_CH = _TOKENS * _TOP_K // _NW


def _tc_body(x_ref, w_ref, logits_ref, packed_ref, gates_ref,
             hist_ref, acc_ref):
    b = pl.program_id(0)

    @pl.when(b == 0)
    def _():
        acc_ref[...] = jnp.zeros_like(acc_ref)

    x = x_ref[...]
    w = w_ref[...]
    logits = lax.dot_general(x, w, (((1,), (1,)), ((), ())),
                             preferred_element_type=jnp.float32)
    logits_ref[...] = logits

    col = lax.broadcasted_iota(jnp.int32, (_BT, _NUM_EXPERTS), 1)
    m0 = jnp.max(logits, axis=1, keepdims=True)
    idx0 = jnp.min(jnp.where(logits == m0, col, _NUM_EXPERTS), axis=1,
                   keepdims=True)
    masked = jnp.where(col == idx0, -jnp.inf, logits)
    m1 = jnp.max(masked, axis=1, keepdims=True)
    idx1 = jnp.min(jnp.where(masked == m1, col, _NUM_EXPERTS), axis=1,
                   keepdims=True)

    s = jnp.exp(m1 - m0)
    inv = 1.0 / (1.0 + s)
    gates_ref[...] = jnp.concatenate([inv, s * inv], axis=1)

    oh0 = (col == idx0).astype(jnp.float32)
    oh1 = (col == idx1).astype(jnp.float32)
    ohs = oh0 + oh1
    ri = lax.broadcasted_iota(jnp.int32, (_BT, _BT), 0)
    ci = lax.broadcasted_iota(jnp.int32, (_BT, _BT), 1)
    tril = (ci < ri).astype(jnp.float32)
    P = lax.dot_general(tril, ohs, (((1,), (0,)), ((), ())),
                        preferred_element_type=jnp.float32)
    acc = acc_ref[0:1, :]
    pa = P + acc
    base0 = jnp.sum(pa * oh0, axis=1, keepdims=True).astype(jnp.int32)
    base1 = jnp.sum(pa * oh1, axis=1, keepdims=True).astype(jnp.int32)
    packed_ref[...] = jnp.concatenate(
        [idx0 * 16384 + base0, idx1 * 16384 + base1], axis=1)

    acc_new = acc + jnp.sum(ohs, axis=0, keepdims=True)
    acc_ref[...] = jnp.broadcast_to(acc_new, acc_ref.shape)
    hist_ref[...] = jnp.broadcast_to(acc_new, hist_ref.shape).astype(jnp.int32)


def _tc_call(x, w):
    out_shapes = [
        jax.ShapeDtypeStruct((_TOKENS, _NUM_EXPERTS), jnp.float32),
        jax.ShapeDtypeStruct((_TOKENS, _TOP_K), jnp.int32),
        jax.ShapeDtypeStruct((_TOKENS, _TOP_K), jnp.float32),
        jax.ShapeDtypeStruct((8, _NUM_EXPERTS), jnp.int32),
    ]
    in_specs = [
        pl.BlockSpec((_BT, _INPUT_SIZE), lambda b: (b, 0)),
        pl.BlockSpec((_NUM_EXPERTS, _INPUT_SIZE), lambda b: (0, 0)),
    ]
    out_specs = [
        pl.BlockSpec((_BT, _NUM_EXPERTS), lambda b: (b, 0)),
        pl.BlockSpec((_BT, _TOP_K), lambda b: (b, 0)),
        pl.BlockSpec((_BT, _TOP_K), lambda b: (b, 0)),
        pl.BlockSpec((8, _NUM_EXPERTS), lambda b: (0, 0)),
    ]
    return pl.pallas_call(
        _tc_body,
        grid=(_NB,),
        in_specs=in_specs,
        out_specs=out_specs,
        out_shape=out_shapes,
        scratch_shapes=[pltpu.VMEM((8, _NUM_EXPERTS), jnp.float32)],
        compiler_params=pltpu.CompilerParams(
            dimension_semantics=("arbitrary",)),
    )(x, w)


_SCH = _TOKENS * _TOP_K // 16


def _sc_body(p_hbm, g_hbm, hist_hbm,
             oi_hbm, ob_hbm, og_hbm,
             hist_v, excl_v, p_v, g_v, dest_v, ivals_v,
             inv_v, ob_v, og_v, inv_sh, sem):
    tid = lax.axis_index("s")
    base = tid * _SCH

    hload = pltpu.make_async_copy(hist_hbm, hist_v, sem)
    pload = pltpu.make_async_copy(p_hbm.at[pl.ds(base, _SCH)], p_v, sem)
    gload = pltpu.make_async_copy(g_hbm, g_v, sem)
    hload.start()
    pload.start()
    gload.start()
    hload.wait()

    carry = jnp.int32(0)
    for j in range(4):
        c = hist_v[pl.ds(j * 16, 16)]
        s = plsc.cumsum(c)
        excl_v[pl.ds(j * 16, 16)] = s - c + carry
        carry = carry + jnp.sum(c)

    lanes = lax.iota(jnp.int32, 16)
    pload.wait()

    def dest_body(i, _):
        for u in range(4):
            off = i * 64 + u * 16
            p = p_v[pl.ds(off, 16)]
            k = lax.shift_right_logical(p, 14)
            r = p & 16383
            dest_v[pl.ds(off, 16)] = plsc.load_gather(excl_v, [k]) + r
            ivals_v[pl.ds(off, 16)] = base + off + lanes
        return 0

    lax.fori_loop(0, _SCH // 64, dest_body, 0)

    pltpu.sync_copy(ivals_v, inv_sh.at[dest_v])
    plsc.subcore_barrier()
    pltpu.sync_copy(inv_sh.at[pl.ds(base, _SCH)], inv_v)
    gload.wait()

    n_mask = _TOKENS * _TOP_K - 1
    for i in range(_SCH // 16):
        off = i * 16
        inv = inv_v[pl.ds(off, 16)]
        ob_v[pl.ds(off, 16)] = inv >> 1
        og_v[pl.ds(off, 16)] = plsc.load_gather(g_v, [inv & n_mask])

    stores = [
        pltpu.make_async_copy(inv_v, oi_hbm.at[pl.ds(base, _SCH)], sem),
        pltpu.make_async_copy(ob_v, ob_hbm.at[pl.ds(base, _SCH)], sem),
        pltpu.make_async_copy(og_v, og_hbm.at[pl.ds(base, _SCH)], sem),
    ]
    for c in stores:
        c.start()
    for c in stores:
        c.wait()


def _sc_call(packed, gates, hist):
    n = _TOKENS * _TOP_K
    mesh = plsc.VectorSubcoreMesh(
        core_axis_name="c", subcore_axis_name="s", num_cores=1)
    f = pl.kernel(
        _sc_body,
        out_type=[
            jax.ShapeDtypeStruct((n,), jnp.int32),
            jax.ShapeDtypeStruct((n,), jnp.int32),
            jax.ShapeDtypeStruct((n,), jnp.float32),
        ],
        mesh=mesh,
        scratch_types=[
            pltpu.VMEM((_NUM_EXPERTS,), jnp.int32),
            pltpu.VMEM((_NUM_EXPERTS,), jnp.int32),
            pltpu.VMEM((_SCH,), jnp.int32),
            pltpu.VMEM((n,), jnp.float32),
            pltpu.VMEM((_SCH,), jnp.int32),
            pltpu.VMEM((_SCH,), jnp.int32),
            pltpu.VMEM((_SCH,), jnp.int32),
            pltpu.VMEM((_SCH,), jnp.int32),
            pltpu.VMEM((_SCH,), jnp.float32),
            pltpu.VMEM_SHARED((n,), jnp.int32),
            pltpu.SemaphoreType.DMA,
        ],
        compiler_params=pltpu.CompilerParams(needs_layout_passes=False),
    )
    return f(packed, gates, hist)


def kernel(hidden_states, W):
    logits, packed, gates, hist8 = _tc_call(hidden_states, W)
    expert_size = hist8[0]
    isorted, bidx, bgates = _sc_call(
        packed.reshape(-1), gates.reshape(-1), expert_size)
    return (isorted, bidx, bgates, expert_size, logits)


# speedup vs baseline: 2.2470x; 1.0016x over previous
import jax
import jax.numpy as jnp
from jax import lax
from jax.experimental import pallas as pl
from jax.experimental.pallas import tpu as pltpu
from jax.experimental.pallas import tpu_sc as plsc

_NUM_EXPERTS = 64
_TOP_K = 2
_INPUT_SIZE = 4096
_TOKENS = 8192
_BT = 1024
_NB = _TOKENS // _BT


def _tc_body(x_ref, w_ref, logits_ref, packed_ref, gates_ref,
             hist_ref, acc_ref):
    b = pl.program_id(0)

    @pl.when(b == 0)
    def _():
        acc_ref[...] = jnp.zeros_like(acc_ref)

    x = x_ref[...]
    w = w_ref[...]
    logits = lax.dot_general(x, w, (((1,), (1,)), ((), ())),
                             preferred_element_type=jnp.float32)
    logits_ref[...] = logits

    col = lax.broadcasted_iota(jnp.int32, (_BT, _NUM_EXPERTS), 1)
    m0 = jnp.max(logits, axis=1, keepdims=True)
    idx0 = jnp.min(jnp.where(logits == m0, col, _NUM_EXPERTS), axis=1,
                   keepdims=True)
    masked = jnp.where(col == idx0, -jnp.inf, logits)
    m1 = jnp.max(masked, axis=1, keepdims=True)
    idx1 = jnp.min(jnp.where(masked == m1, col, _NUM_EXPERTS), axis=1,
                   keepdims=True)

    s = jnp.exp(m1 - m0)
    inv = 1.0 / (1.0 + s)
    gates_ref[...] = jnp.concatenate([inv, s * inv], axis=1)

    oh0 = (col == idx0).astype(jnp.float32)
    oh1 = (col == idx1).astype(jnp.float32)
    ohs = oh0 + oh1
    ri = lax.broadcasted_iota(jnp.int32, (_BT, _BT), 0)
    ci = lax.broadcasted_iota(jnp.int32, (_BT, _BT), 1)
    tril = (ci < ri).astype(jnp.float32)
    P = lax.dot_general(tril, ohs, (((1,), (0,)), ((), ())),
                        preferred_element_type=jnp.float32)
    acc = acc_ref[0:1, :]
    pa = P + acc
    base0 = jnp.sum(pa * oh0, axis=1, keepdims=True).astype(jnp.int32)
    base1 = jnp.sum(pa * oh1, axis=1, keepdims=True).astype(jnp.int32)
    packed_ref[...] = jnp.concatenate(
        [idx0 * 16384 + base0, idx1 * 16384 + base1], axis=1)

    acc_new = acc + jnp.sum(ohs, axis=0, keepdims=True)
    acc_ref[...] = jnp.broadcast_to(acc_new, acc_ref.shape)
    hist_ref[...] = jnp.broadcast_to(acc_new, hist_ref.shape).astype(jnp.int32)


def _tc_call(x, w):
    out_shapes = [
        jax.ShapeDtypeStruct((_TOKENS, _NUM_EXPERTS), jnp.float32),
        jax.ShapeDtypeStruct((_TOKENS, _TOP_K), jnp.int32),
        jax.ShapeDtypeStruct((_TOKENS, _TOP_K), jnp.float32),
        jax.ShapeDtypeStruct((8, _NUM_EXPERTS), jnp.int32),
    ]
    in_specs = [
        pl.BlockSpec((_BT, _INPUT_SIZE), lambda b: (b, 0)),
        pl.BlockSpec((_NUM_EXPERTS, _INPUT_SIZE), lambda b: (0, 0)),
    ]
    out_specs = [
        pl.BlockSpec((_BT, _NUM_EXPERTS), lambda b: (b, 0)),
        pl.BlockSpec((_BT, _TOP_K), lambda b: (b, 0)),
        pl.BlockSpec((_BT, _TOP_K), lambda b: (b, 0)),
        pl.BlockSpec((8, _NUM_EXPERTS), lambda b: (0, 0)),
    ]
    return pl.pallas_call(
        _tc_body,
        grid=(_NB,),
        in_specs=in_specs,
        out_specs=out_specs,
        out_shape=out_shapes,
        scratch_shapes=[pltpu.VMEM((8, _NUM_EXPERTS), jnp.float32)],
        compiler_params=pltpu.CompilerParams(
            dimension_semantics=("arbitrary",)),
    )(x, w)


_SCH = _TOKENS * _TOP_K // 16


def _sc_body(p_hbm, g_hbm, hist_hbm,
             oi_hbm, ob_hbm, og_hbm,
             hist_v, excl_v, p_v, g_v, dest_v, ivals_v,
             inv_v, ob_v, og_v, inv_sh, sem):
    tid = lax.axis_index("s")
    base = tid * _SCH

    hload = pltpu.make_async_copy(hist_hbm, hist_v, sem)
    pload = pltpu.make_async_copy(p_hbm.at[pl.ds(base, _SCH)], p_v, sem)
    gload = pltpu.make_async_copy(g_hbm, g_v, sem)
    hload.start()
    pload.start()
    gload.start()
    hload.wait()

    carry = jnp.int32(0)
    for j in range(4):
        c = hist_v[pl.ds(j * 16, 16)]
        s = plsc.cumsum(c)
        excl_v[pl.ds(j * 16, 16)] = s - c + carry
        carry = carry + jnp.sum(c)

    lanes = lax.iota(jnp.int32, 16)
    pload.wait()

    def dest_body(i, _):
        for u in range(4):
            off = i * 64 + u * 16
            p = p_v[pl.ds(off, 16)]
            k = lax.shift_right_logical(p, 14)
            r = p & 16383
            dest_v[pl.ds(off, 16)] = plsc.load_gather(excl_v, [k]) + r
            ivals_v[pl.ds(off, 16)] = base + off + lanes
        return 0

    lax.fori_loop(0, _SCH // 64, dest_body, 0)

    pltpu.sync_copy(ivals_v, inv_sh.at[dest_v])
    plsc.subcore_barrier()
    pltpu.sync_copy(inv_sh.at[pl.ds(base, _SCH)], inv_v)
    gload.wait()

    n_mask = _TOKENS * _TOP_K - 1
    for i in range(_SCH // 16):
        off = i * 16
        inv = inv_v[pl.ds(off, 16)]
        ob_v[pl.ds(off, 16)] = inv >> 1
        og_v[pl.ds(off, 16)] = plsc.load_gather(g_v, [inv & n_mask])

    stores = [
        pltpu.make_async_copy(inv_v, oi_hbm.at[pl.ds(base, _SCH)], sem),
        pltpu.make_async_copy(ob_v, ob_hbm.at[pl.ds(base, _SCH)], sem),
        pltpu.make_async_copy(og_v, og_hbm.at[pl.ds(base, _SCH)], sem),
    ]
    for c in stores:
        c.start()
    for c in stores:
        c.wait()


def _sc_call(packed, gates, hist):
    n = _TOKENS * _TOP_K
    mesh = plsc.VectorSubcoreMesh(
        core_axis_name="c", subcore_axis_name="s", num_cores=1)
    f = pl.kernel(
        _sc_body,
        out_type=[
            jax.ShapeDtypeStruct((n,), jnp.int32),
            jax.ShapeDtypeStruct((n,), jnp.int32),
            jax.ShapeDtypeStruct((n,), jnp.float32),
        ],
        mesh=mesh,
        scratch_types=[
            pltpu.VMEM((_NUM_EXPERTS,), jnp.int32),
            pltpu.VMEM((_NUM_EXPERTS,), jnp.int32),
            pltpu.VMEM((_SCH,), jnp.int32),
            pltpu.VMEM((n,), jnp.float32),
            pltpu.VMEM((_SCH,), jnp.int32),
            pltpu.VMEM((_SCH,), jnp.int32),
            pltpu.VMEM((_SCH,), jnp.int32),
            pltpu.VMEM((_SCH,), jnp.int32),
            pltpu.VMEM((_SCH,), jnp.float32),
            pltpu.VMEM_SHARED((n,), jnp.int32),
            pltpu.SemaphoreType.DMA,
        ],
        compiler_params=pltpu.CompilerParams(needs_layout_passes=False),
    )
    return f(packed, gates, hist)


def kernel(hidden_states, W):
    logits, packed, gates, hist8 = _tc_call(hidden_states, W)
    expert_size = hist8[0]
    isorted, bidx, bgates = _sc_call(
        packed.reshape(-1), gates.reshape(-1), expert_size)
    return (isorted, bidx, bgates, expert_size, logits)
